# Initial kernel scaffold; baseline (speedup 1.0000x reference)
import jax
import jax.numpy as jnp
from jax.experimental import pallas as pl


def kernel(x, edge_index, W0_src, W0_dst, att0, b0, W1_src, W1_dst, att1, b1, Wp1, bp1, Wp2, bp2):
    raise NotImplementedError("write your pallas kernel here")


# speedup vs baseline: 8.1218x; 8.1218x over previous
import functools

import jax
import jax.numpy as jnp
from jax import lax
from jax.experimental import pallas as pl
from jax.experimental.pallas import tpu as pltpu
from jax.experimental.pallas import tpu_sc as plsc

_NC = 2
_NS = 16
_L = 16

_GDN = lax.GatherDimensionNumbers(
    offset_dims=(), collapsed_slice_dims=(0,), start_index_map=(0,))


def _lane_perm(v, idx):
    return lax.gather(v, idx[:, None], _GDN, (1,),
                      mode=lax.GatherScatterMode.PROMISE_IN_BOUNDS)


def _lane_sum(v):
    iota = lax.broadcasted_iota(jnp.int32, (_L,), 0)
    for step in (1, 2, 4, 8):
        v = v + _lane_perm(v, iota ^ step)
    return v


def _proj2_body(x_ref, wa_ref, wb_ref, oa_ref, ob_ref):
    x = x_ref[...]
    oa_ref[...] = jnp.dot(x, wa_ref[...], preferred_element_type=jnp.float32)
    ob_ref[...] = jnp.dot(x, wb_ref[...], preferred_element_type=jnp.float32)


def _proj2(x, wa, wb, blk):
    n, d = x.shape
    h = wa.shape[1]
    grid = (n + blk - 1) // blk
    return pl.pallas_call(
        _proj2_body,
        grid=(grid,),
        in_specs=[
            pl.BlockSpec((blk, d), lambda i: (i, 0)),
            pl.BlockSpec((d, h), lambda i: (0, 0)),
            pl.BlockSpec((d, h), lambda i: (0, 0)),
        ],
        out_specs=[
            pl.BlockSpec((blk, h), lambda i: (i, 0)),
            pl.BlockSpec((blk, h), lambda i: (i, 0)),
        ],
        out_shape=[
            jax.ShapeDtypeStruct((n, h), jnp.float32),
            jax.ShapeDtypeStruct((n, h), jnp.float32),
        ],
    )(x, wa, wb)


def _norm_h(acc_ref, den_ref, b_ref):
    ssum = acc_ref[0] + acc_ref[1]
    den = den_ref[0, :, 0:1] + den_ref[1, :, 0:1]
    den = jnp.where(den > 0.0, den, 1.0)
    return jnp.maximum(ssum / den + b_ref[...], 0.0)


def _epi_proj2_body(acc_ref, den_ref, b_ref, wa_ref, wb_ref, oa_ref, ob_ref):
    h = _norm_h(acc_ref, den_ref, b_ref)
    oa_ref[...] = jnp.dot(h, wa_ref[...], preferred_element_type=jnp.float32)
    ob_ref[...] = jnp.dot(h, wb_ref[...], preferred_element_type=jnp.float32)


def _epi_proj2(acc, den, b, wa, wb, blk):
    _, n, hdim = acc.shape
    ho = wa.shape[1]
    grid = (n + blk - 1) // blk
    return pl.pallas_call(
        _epi_proj2_body,
        grid=(grid,),
        in_specs=[
            pl.BlockSpec((2, blk, hdim), lambda i: (0, i, 0)),
            pl.BlockSpec((2, blk, _L), lambda i: (0, i, 0)),
            pl.BlockSpec((1, hdim), lambda i: (0, 0)),
            pl.BlockSpec((hdim, ho), lambda i: (0, 0)),
            pl.BlockSpec((hdim, ho), lambda i: (0, 0)),
        ],
        out_specs=[
            pl.BlockSpec((blk, ho), lambda i: (i, 0)),
            pl.BlockSpec((blk, ho), lambda i: (i, 0)),
        ],
        out_shape=[
            jax.ShapeDtypeStruct((n, ho), jnp.float32),
            jax.ShapeDtypeStruct((n, ho), jnp.float32),
        ],
    )(acc, den, b, wa, wb)


def _final_body(acc_ref, den_ref, b_ref, wp1_ref, bp1_ref, wp2_ref, bp2_ref,
                o_ref):
    h = _norm_h(acc_ref, den_ref, b_ref)
    t = jnp.dot(h, wp1_ref[...], preferred_element_type=jnp.float32)
    t = t + bp1_ref[...]
    o = jnp.dot(t, wp2_ref[...], preferred_element_type=jnp.float32)
    o_ref[...] = o + bp2_ref[...]


def _final(acc, den, b, wp1, bp1, wp2, bp2, blk):
    _, n, hdim = acc.shape
    hmid = wp1.shape[1]
    odim = wp2.shape[1]
    grid = (n + blk - 1) // blk
    return pl.pallas_call(
        _final_body,
        grid=(grid,),
        in_specs=[
            pl.BlockSpec((2, blk, hdim), lambda i: (0, i, 0)),
            pl.BlockSpec((2, blk, _L), lambda i: (0, i, 0)),
            pl.BlockSpec((1, hdim), lambda i: (0, 0)),
            pl.BlockSpec((hdim, hmid), lambda i: (0, 0)),
            pl.BlockSpec((1, hmid), lambda i: (0, 0)),
            pl.BlockSpec((hmid, odim), lambda i: (0, 0)),
            pl.BlockSpec((1, odim), lambda i: (0, 0)),
        ],
        out_specs=pl.BlockSpec((blk, odim), lambda i: (i, 0)),
        out_shape=jax.ShapeDtypeStruct((n, odim), jnp.float32),
    )(acc, den, b, wp1, bp1, wp2, bp2)


@functools.lru_cache(maxsize=None)
def _make_sc_layer(n, e, hdim):
    nw = _NC * _NS
    epw = e // nw
    chunk = 8
    for c in range(128, 7, -8):
        if epw % c == 0:
            chunk = c
            break
    nch = epw // chunk
    kk = hdim // _L
    nden = -(-n // 8)
    rtot = -(-(n + nden) // (16 * _NS)) * 16 * _NS
    rps = rtot // _NS
    nblk = rps // 16

    mesh = plsc.VectorSubcoreMesh(core_axis_name="c", subcore_axis_name="s")

    def body(xl_hbm, xr_hbm, src_hbm, dst_hbm, att_hbm,
             raw_out,
             srcv, dstv, dstv8, xlv, xrv, denv, attv, bounce, accs,
             sem_a, sem_b):
        c = lax.axis_index("c")
        s = lax.axis_index("s")
        wid = c * _NS + s
        r0 = pl.multiple_of(s * rps, 8)

        zt = jnp.zeros((_L,), jnp.float32)
        for r in range(16):
            for k in range(kk):
                bounce[r, pl.ds(k * _L, _L)] = zt

        def zero_body(j, carry):
            rr = pl.multiple_of(r0 + j * 16, 8)
            pltpu.sync_copy(bounce, accs.at[pl.ds(rr, 16)])
            return carry

        lax.fori_loop(0, nblk, zero_body, 0)
        pltpu.sync_copy(att_hbm, attv)
        plsc.subcore_barrier()

        att_regs = [attv[pl.ds(k * _L, _L)] for k in range(kk)]
        lane0 = lax.broadcasted_iota(jnp.int32, (_L,), 0) == 0

        def chunk_body(ch, carry):
            base = pl.multiple_of(wid * epw + ch * chunk, 8)
            pltpu.sync_copy(src_hbm.at[pl.ds(base, chunk)], srcv)
            pltpu.sync_copy(dst_hbm.at[pl.ds(base, chunk)], dstv)
            cp1 = pltpu.async_copy(xl_hbm.at[srcv], xlv, sem_a)
            cp2 = pltpu.async_copy(xr_hbm.at[dstv], xrv, sem_b)
            for t in range(chunk // _L):
                d = dstv[pl.ds(t * _L, _L)]
                dstv8[pl.ds(t * _L, _L)] = n + lax.shift_right_logical(d, 3)
            cp1.wait()
            cp2.wait()

            def edge_body(i, ecarry):
                xj = [xlv[i, pl.ds(k * _L, _L)] for k in range(kk)]
                acc = None
                for k in range(kk):
                    z = xrv[i, pl.ds(k * _L, _L)] + xj[k]
                    lr = jnp.maximum(z, 0.0) + 0.2 * jnp.minimum(z, 0.0)
                    t = lr * att_regs[k]
                    acc = t if acc is None else acc + t
                w = jnp.exp(_lane_sum(acc))
                w0 = jnp.where(lane0, w, 0.0)
                grp = dstv[pl.ds(pl.multiple_of((i >> 4) << 4, 16), _L)]
                dvec = _lane_perm(
                    grp, jnp.full((_L,), 0, jnp.int32) + (i & 15))
                tgtf = lax.rem(dvec, 8).astype(jnp.float32)
                for k in range(8):
                    eqf = jnp.maximum(1.0 - jnp.abs(tgtf - float(k)), 0.0)
                    denv[i, pl.ds(k * _L, _L)] = w0 * eqf
                for k in range(kk):
                    xlv[i, pl.ds(k * _L, _L)] = xj[k] * w
                return ecarry

            lax.fori_loop(0, chunk, edge_body, 0)
            pltpu.sync_copy(xlv, accs.at[dstv], add=True)
            pltpu.sync_copy(denv, accs.at[dstv8], add=True)
            return carry

        lax.fori_loop(0, nch, chunk_body, 0)
        plsc.subcore_barrier()

        def out_body(j, carry):
            rr = pl.multiple_of(r0 + j * 16, 8)
            pltpu.sync_copy(accs.at[pl.ds(rr, 16)], bounce)
            pltpu.sync_copy(bounce, raw_out.at[c, pl.ds(rr, 16)])
            return carry

        lax.fori_loop(0, nblk, out_body, 0)

    kern = pl.kernel(
        body,
        out_type=jax.ShapeDtypeStruct((_NC, rtot, hdim), jnp.float32),
        mesh=mesh,
        scratch_types=[
            pltpu.VMEM((chunk,), jnp.int32),
            pltpu.VMEM((chunk,), jnp.int32),
            pltpu.VMEM((chunk,), jnp.int32),
            pltpu.VMEM((chunk, hdim), jnp.float32),
            pltpu.VMEM((chunk, hdim), jnp.float32),
            pltpu.VMEM((chunk, hdim), jnp.float32),
            pltpu.VMEM((hdim,), jnp.float32),
            pltpu.VMEM((16, hdim), jnp.float32),
            pltpu.VMEM_SHARED((rtot, hdim), jnp.float32),
            pltpu.SemaphoreType.DMA,
            pltpu.SemaphoreType.DMA,
        ],
    )
    return kern, nden


def _sc_layer(xl, xr, src, dst, att):
    n, hdim = xl.shape
    e = src.shape[0]
    kern, nden = _make_sc_layer(n, e, hdim)
    raw = kern(xl, xr, src, dst, att)
    acc = raw[:, :n]
    den = raw[:, n:n + nden].reshape(_NC, nden * 8, _L)[:, :n]
    return acc, den


def kernel(x, edge_index, W0_src, W0_dst, att0, b0, W1_src, W1_dst, att1, b1,
           Wp1, bp1, Wp2, bp2):
    n, _ = x.shape
    src = edge_index[0]
    dst = edge_index[1]
    blk = 1000

    xl0, xr0 = _proj2(x, W0_src, W0_dst, blk)
    acc0, den0 = _sc_layer(xl0, xr0, src, dst, att0)
    xl1, xr1 = _epi_proj2(acc0, den0, b0.reshape(1, -1), W1_src, W1_dst, blk)
    acc1, den1 = _sc_layer(xl1, xr1, src, dst, att1)
    out = _final(acc1, den1, b1.reshape(1, -1), Wp1, bp1.reshape(1, -1),
                 Wp2, bp2.reshape(1, -1), blk)
    return out


# speedup vs baseline: 9.7706x; 1.2030x over previous
import functools

import jax
import jax.numpy as jnp
from jax import lax
from jax.experimental import pallas as pl
from jax.experimental.pallas import tpu as pltpu
from jax.experimental.pallas import tpu_sc as plsc

_NC = 2
_NS = 16
_L = 16

_GDN = lax.GatherDimensionNumbers(
    offset_dims=(), collapsed_slice_dims=(0,), start_index_map=(0,))


def _lane_perm(v, idx):
    return lax.gather(v, idx[:, None], _GDN, (1,),
                      mode=lax.GatherScatterMode.PROMISE_IN_BOUNDS)


def _lane_sum(v):
    iota = lax.broadcasted_iota(jnp.int32, (_L,), 0)
    for step in (1, 2, 4, 8):
        v = v + _lane_perm(v, iota ^ step)
    return v


def _proj2_body(x_ref, wa_ref, wb_ref, oa_ref, ob_ref):
    x = x_ref[...]
    oa_ref[...] = jnp.dot(x, wa_ref[...], preferred_element_type=jnp.float32)
    ob_ref[...] = jnp.dot(x, wb_ref[...], preferred_element_type=jnp.float32)


def _proj2(x, wa, wb, blk):
    n, d = x.shape
    h = wa.shape[1]
    grid = (n + blk - 1) // blk
    return pl.pallas_call(
        _proj2_body,
        grid=(grid,),
        in_specs=[
            pl.BlockSpec((blk, d), lambda i: (i, 0)),
            pl.BlockSpec((d, h), lambda i: (0, 0)),
            pl.BlockSpec((d, h), lambda i: (0, 0)),
        ],
        out_specs=[
            pl.BlockSpec((blk, h), lambda i: (i, 0)),
            pl.BlockSpec((blk, h), lambda i: (i, 0)),
        ],
        out_shape=[
            jax.ShapeDtypeStruct((n, h), jnp.float32),
            jax.ShapeDtypeStruct((n, h), jnp.float32),
        ],
    )(x, wa, wb)


def _norm_h(acc_ref, den_ref, b_ref):
    ssum = acc_ref[0] + acc_ref[1]
    den = den_ref[0] + den_ref[1]
    den = jnp.where(den > 0.0, den, 1.0)
    return jnp.maximum(ssum / den + b_ref[...], 0.0)


def _epi_proj2_body(acc_ref, den_ref, b_ref, wa_ref, wb_ref, oa_ref, ob_ref):
    h = _norm_h(acc_ref, den_ref, b_ref)
    oa_ref[...] = jnp.dot(h, wa_ref[...], preferred_element_type=jnp.float32)
    ob_ref[...] = jnp.dot(h, wb_ref[...], preferred_element_type=jnp.float32)


def _epi_proj2(acc, den, b, wa, wb, blk):
    _, n, hdim = acc.shape
    ho = wa.shape[1]
    grid = (n + blk - 1) // blk
    return pl.pallas_call(
        _epi_proj2_body,
        grid=(grid,),
        in_specs=[
            pl.BlockSpec((2, blk, hdim), lambda i: (0, i, 0)),
            pl.BlockSpec((2, blk, 1), lambda i: (0, i, 0)),
            pl.BlockSpec((1, hdim), lambda i: (0, 0)),
            pl.BlockSpec((hdim, ho), lambda i: (0, 0)),
            pl.BlockSpec((hdim, ho), lambda i: (0, 0)),
        ],
        out_specs=[
            pl.BlockSpec((blk, ho), lambda i: (i, 0)),
            pl.BlockSpec((blk, ho), lambda i: (i, 0)),
        ],
        out_shape=[
            jax.ShapeDtypeStruct((n, ho), jnp.float32),
            jax.ShapeDtypeStruct((n, ho), jnp.float32),
        ],
    )(acc, den, b, wa, wb)


def _final_body(acc_ref, den_ref, b_ref, wp1_ref, bp1_ref, wp2_ref, bp2_ref,
                o_ref):
    h = _norm_h(acc_ref, den_ref, b_ref)
    t = jnp.dot(h, wp1_ref[...], preferred_element_type=jnp.float32)
    t = t + bp1_ref[...]
    o = jnp.dot(t, wp2_ref[...], preferred_element_type=jnp.float32)
    o_ref[...] = o + bp2_ref[...]


def _final(acc, den, b, wp1, bp1, wp2, bp2, blk):
    _, n, hdim = acc.shape
    hmid = wp1.shape[1]
    odim = wp2.shape[1]
    grid = (n + blk - 1) // blk
    return pl.pallas_call(
        _final_body,
        grid=(grid,),
        in_specs=[
            pl.BlockSpec((2, blk, hdim), lambda i: (0, i, 0)),
            pl.BlockSpec((2, blk, 1), lambda i: (0, i, 0)),
            pl.BlockSpec((1, hdim), lambda i: (0, 0)),
            pl.BlockSpec((hdim, hmid), lambda i: (0, 0)),
            pl.BlockSpec((1, hmid), lambda i: (0, 0)),
            pl.BlockSpec((hmid, odim), lambda i: (0, 0)),
            pl.BlockSpec((1, odim), lambda i: (0, 0)),
        ],
        out_specs=pl.BlockSpec((blk, odim), lambda i: (i, 0)),
        out_shape=jax.ShapeDtypeStruct((n, odim), jnp.float32),
    )(acc, den, b, wp1, bp1, wp2, bp2)


@functools.lru_cache(maxsize=None)
def _make_sc_layer(n, e, hdim):
    nw = _NC * _NS
    epw = e // nw
    chunk = 64
    nch = epw // chunk
    tail = epw - nch * chunk
    kk = hdim // _L
    nden = -(-n // hdim)
    rtot = -(-(n + nden) // (16 * _NS)) * 16 * _NS
    rps = rtot // _NS
    nblk = rps // 16

    mesh = plsc.VectorSubcoreMesh(core_axis_name="c", subcore_axis_name="s")

    def body(xl_hbm, xr_hbm, src_hbm, dst_hbm, att_hbm,
             raw_out,
             srcv, dstv, dstv8, srcvt, dstvt, dstv8t, xlv, xrv, denv, attv,
             accs, sem_g, sem_g2, sem_t):
        c = lax.axis_index("c")
        s = lax.axis_index("s")
        wid = c * _NS + s
        r0 = pl.multiple_of(s * rps, 8)
        bounce = denv.at[pl.ds(0, 16)]

        zt = jnp.zeros((_L,), jnp.float32)
        for r in range(16):
            for k in range(kk):
                denv[r, pl.ds(k * _L, _L)] = zt

        def zero_body(j, carry):
            rr = pl.multiple_of(r0 + j * 16, 8)
            pltpu.sync_copy(bounce, accs.at[pl.ds(rr, 16)])
            return carry

        lax.fori_loop(0, nblk, zero_body, 0)
        pltpu.sync_copy(att_hbm, attv)
        plsc.subcore_barrier()

        att_regs = [attv[pl.ds(k * _L, _L)] for k in range(kk)]
        lane0 = lax.broadcasted_iota(jnp.int32, (_L,), 0) == 0
        iota16 = lax.broadcasted_iota(jnp.int32, (_L,), 0)
        laneid = [(k * _L + iota16).astype(jnp.float32) for k in range(kk)]

        def load_idx(ch, b):
            base = pl.multiple_of(wid * epw + ch * chunk, 8)
            pltpu.sync_copy(src_hbm.at[pl.ds(base, chunk)], srcv.at[b])
            pltpu.sync_copy(dst_hbm.at[pl.ds(base, chunk)], dstv.at[b])

        def start_gather(b):
            sem = sem_g if b == 0 else sem_g2
            cp1 = pltpu.async_copy(xl_hbm.at[srcv.at[b]], xlv.at[b], sem)
            cp2 = pltpu.async_copy(xr_hbm.at[dstv.at[b]], xrv.at[b], sem)
            return cp1, cp2

        def edge_block(xlv_b, xrv_b, denv_b, dstv_b, dstv8_b, sz):
            for t in range(sz // _L):
                d = dstv_b[pl.ds(t * _L, _L)]
                dstv8_b[pl.ds(t * _L, _L)] = (
                    n + lax.shift_right_logical(d, 7))

            def edge_body(i, ecarry):
                xj = [xlv_b[i, pl.ds(k * _L, _L)] for k in range(kk)]
                acc = None
                for k in range(kk):
                    z = xrv_b[i, pl.ds(k * _L, _L)] + xj[k]
                    lr = jnp.maximum(z, 0.0) + 0.2 * jnp.minimum(z, 0.0)
                    t = lr * att_regs[k]
                    acc = t if acc is None else acc + t
                w = jnp.exp(_lane_sum(acc))
                w0 = jnp.where(lane0, w, 0.0)
                grp = dstv_b[pl.ds(pl.multiple_of((i >> 4) << 4, 16), _L)]
                dvec = _lane_perm(
                    grp, jnp.full((_L,), 0, jnp.int32) + (i & 15))
                tgtf = lax.rem(dvec, hdim).astype(jnp.float32)
                for k in range(kk):
                    eqf = jnp.maximum(1.0 - jnp.abs(laneid[k] - tgtf), 0.0)
                    denv_b[i, pl.ds(k * _L, _L)] = w * eqf
                for k in range(kk):
                    xlv_b[i, pl.ds(k * _L, _L)] = xj[k] * w
                return ecarry

            lax.fori_loop(0, sz, edge_body, 0)

        def compute_scatter(b, cps):
            cps[0].wait()
            cps[1].wait()
            edge_block(xlv.at[b], xrv.at[b], denv, dstv.at[b], dstv8.at[b],
                       chunk)
            pltpu.sync_copy(xlv.at[b], accs.at[dstv.at[b]], add=True)
            pltpu.sync_copy(denv, accs.at[dstv8.at[b]], add=True)

        if nch >= 3:
            s0 = 1 if nch % 2 == 0 else 0
            if s0:
                load_idx(0, 0)
                cps0 = start_gather(0)
                compute_scatter(0, cps0)
            load_idx(s0, 0)
            cps_a = start_gather(0)

            def pair_body(j, carry):
                ch0 = s0 + 2 * j
                load_idx(ch0 + 1, 1)
                cps_b = start_gather(1)
                compute_scatter(0, cps_a)
                load_idx(ch0 + 2, 0)
                cps_a2 = start_gather(0)
                compute_scatter(1, cps_b)
                return carry

            lax.fori_loop(0, (nch - s0 - 1) // 2, pair_body, 0)
            compute_scatter(0, cps_a)
        else:
            def chunk_body(ch, carry):
                load_idx(ch, 0)
                cps = start_gather(0)
                compute_scatter(0, cps)
                return carry

            lax.fori_loop(0, nch, chunk_body, 0)

        if tail:
            base = pl.multiple_of(wid * epw + nch * chunk, 8)
            pltpu.sync_copy(src_hbm.at[pl.ds(base, tail)], srcvt.at[0])
            pltpu.sync_copy(dst_hbm.at[pl.ds(base, tail)], dstvt.at[0])
            xlv_t = xlv.at[0, pl.ds(0, tail)]
            xrv_t = xrv.at[0, pl.ds(0, tail)]
            denv_t = denv.at[pl.ds(0, tail)]
            cp1 = pltpu.async_copy(xl_hbm.at[srcvt.at[0]], xlv_t, sem_t)
            cp2 = pltpu.async_copy(xr_hbm.at[dstvt.at[0]], xrv_t, sem_t)
            cp1.wait()
            cp2.wait()
            edge_block(xlv_t, xrv_t, denv_t, dstvt.at[0], dstv8t.at[0],
                       tail)
            pltpu.sync_copy(xlv_t, accs.at[dstvt.at[0]], add=True)
            pltpu.sync_copy(denv_t, accs.at[dstv8t.at[0]], add=True)

        plsc.subcore_barrier()

        def out_body(j, carry):
            rr = pl.multiple_of(r0 + j * 16, 8)
            pltpu.sync_copy(accs.at[pl.ds(rr, 16)], bounce)
            pltpu.sync_copy(bounce, raw_out.at[c, pl.ds(rr, 16)])
            return carry

        lax.fori_loop(0, nblk, out_body, 0)

    kern = pl.kernel(
        body,
        out_type=jax.ShapeDtypeStruct((_NC, rtot, hdim), jnp.float32),
        mesh=mesh,
        scratch_types=[
            pltpu.VMEM((2, chunk), jnp.int32),
            pltpu.VMEM((2, chunk), jnp.int32),
            pltpu.VMEM((2, chunk), jnp.int32),
            pltpu.VMEM((1, max(tail, 16)), jnp.int32),
            pltpu.VMEM((1, max(tail, 16)), jnp.int32),
            pltpu.VMEM((1, max(tail, 16)), jnp.int32),
            pltpu.VMEM((2, chunk, hdim), jnp.float32),
            pltpu.VMEM((2, chunk, hdim), jnp.float32),
            pltpu.VMEM((chunk, hdim), jnp.float32),
            pltpu.VMEM((hdim,), jnp.float32),
            pltpu.VMEM_SHARED((rtot, hdim), jnp.float32),
            pltpu.SemaphoreType.DMA,
            pltpu.SemaphoreType.DMA,
            pltpu.SemaphoreType.DMA,
        ],
    )
    return kern, nden


def _sc_layer(xl, xr, src, dst, att):
    n, hdim = xl.shape
    e = src.shape[0]
    kern, nden = _make_sc_layer(n, e, hdim)
    raw = kern(xl, xr, src, dst, att)
    acc = raw[:, :n]
    den = raw[:, n:n + nden].reshape(_NC, nden * hdim)[:, :n, None]
    return acc, den


def kernel(x, edge_index, W0_src, W0_dst, att0, b0, W1_src, W1_dst, att1, b1,
           Wp1, bp1, Wp2, bp2):
    n, _ = x.shape
    src = edge_index[0]
    dst = edge_index[1]
    blk = 1000

    xl0, xr0 = _proj2(x, W0_src, W0_dst, blk)
    acc0, den0 = _sc_layer(xl0, xr0, src, dst, att0)
    xl1, xr1 = _epi_proj2(acc0, den0, b0.reshape(1, -1), W1_src, W1_dst, blk)
    acc1, den1 = _sc_layer(xl1, xr1, src, dst, att1)
    out = _final(acc1, den1, b1.reshape(1, -1), Wp1, bp1.reshape(1, -1),
                 Wp2, bp2.reshape(1, -1), blk)
    return out


# speedup vs baseline: 11.2375x; 1.1501x over previous
import functools

import jax
import jax.numpy as jnp
from jax import lax
from jax.experimental import pallas as pl
from jax.experimental.pallas import tpu as pltpu
from jax.experimental.pallas import tpu_sc as plsc

_NC = 2
_NS = 16
_L = 16

_GDN = lax.GatherDimensionNumbers(
    offset_dims=(), collapsed_slice_dims=(0,), start_index_map=(0,))


def _lane_perm(v, idx):
    return lax.gather(v, idx[:, None], _GDN, (1,),
                      mode=lax.GatherScatterMode.PROMISE_IN_BOUNDS)


def _lane_sum(v):
    iota = lax.broadcasted_iota(jnp.int32, (_L,), 0)
    for step in (1, 2, 4, 8):
        v = v + _lane_perm(v, iota ^ step)
    return v


def _proj2_body(x_ref, wa_ref, wb_ref, oa_ref, ob_ref):
    x = x_ref[...]
    oa_ref[...] = jnp.dot(x, wa_ref[...], preferred_element_type=jnp.float32)
    ob_ref[...] = jnp.dot(x, wb_ref[...], preferred_element_type=jnp.float32)


def _proj2(x, wa, wb, blk):
    n, d = x.shape
    h = wa.shape[1]
    grid = (n + blk - 1) // blk
    return pl.pallas_call(
        _proj2_body,
        grid=(grid,),
        in_specs=[
            pl.BlockSpec((blk, d), lambda i: (i, 0)),
            pl.BlockSpec((d, h), lambda i: (0, 0)),
            pl.BlockSpec((d, h), lambda i: (0, 0)),
        ],
        out_specs=[
            pl.BlockSpec((blk, h), lambda i: (i, 0)),
            pl.BlockSpec((blk, h), lambda i: (i, 0)),
        ],
        out_shape=[
            jax.ShapeDtypeStruct((n, h), jnp.float32),
            jax.ShapeDtypeStruct((n, h), jnp.float32),
        ],
    )(x, wa, wb)


def _norm_h(acc_ref, den_ref, b_ref):
    ssum = acc_ref[0] + acc_ref[1]
    den = den_ref[0] + den_ref[1]
    den = jnp.where(den > 0.0, den, 1.0)
    return jnp.maximum(ssum / den + b_ref[...], 0.0)


def _epi_proj2_body(acc_ref, den_ref, b_ref, wa_ref, wb_ref, oa_ref, ob_ref):
    h = _norm_h(acc_ref, den_ref, b_ref)
    oa_ref[...] = jnp.dot(h, wa_ref[...], preferred_element_type=jnp.float32)
    ob_ref[...] = jnp.dot(h, wb_ref[...], preferred_element_type=jnp.float32)


def _epi_proj2(acc, den, b, wa, wb, blk):
    _, n, hdim = acc.shape
    ho = wa.shape[1]
    grid = (n + blk - 1) // blk
    return pl.pallas_call(
        _epi_proj2_body,
        grid=(grid,),
        in_specs=[
            pl.BlockSpec((2, blk, hdim), lambda i: (0, i, 0)),
            pl.BlockSpec((2, blk, 1), lambda i: (0, i, 0)),
            pl.BlockSpec((1, hdim), lambda i: (0, 0)),
            pl.BlockSpec((hdim, ho), lambda i: (0, 0)),
            pl.BlockSpec((hdim, ho), lambda i: (0, 0)),
        ],
        out_specs=[
            pl.BlockSpec((blk, ho), lambda i: (i, 0)),
            pl.BlockSpec((blk, ho), lambda i: (i, 0)),
        ],
        out_shape=[
            jax.ShapeDtypeStruct((n, ho), jnp.float32),
            jax.ShapeDtypeStruct((n, ho), jnp.float32),
        ],
    )(acc, den, b, wa, wb)


def _final_body(acc_ref, den_ref, b_ref, wp1_ref, bp1_ref, wp2_ref, bp2_ref,
                o_ref):
    h = _norm_h(acc_ref, den_ref, b_ref)
    t = jnp.dot(h, wp1_ref[...], preferred_element_type=jnp.float32)
    t = t + bp1_ref[...]
    o = jnp.dot(t, wp2_ref[...], preferred_element_type=jnp.float32)
    o_ref[...] = o + bp2_ref[...]


def _final(acc, den, b, wp1, bp1, wp2, bp2, blk):
    _, n, hdim = acc.shape
    hmid = wp1.shape[1]
    odim = wp2.shape[1]
    grid = (n + blk - 1) // blk
    return pl.pallas_call(
        _final_body,
        grid=(grid,),
        in_specs=[
            pl.BlockSpec((2, blk, hdim), lambda i: (0, i, 0)),
            pl.BlockSpec((2, blk, 1), lambda i: (0, i, 0)),
            pl.BlockSpec((1, hdim), lambda i: (0, 0)),
            pl.BlockSpec((hdim, hmid), lambda i: (0, 0)),
            pl.BlockSpec((1, hmid), lambda i: (0, 0)),
            pl.BlockSpec((hmid, odim), lambda i: (0, 0)),
            pl.BlockSpec((1, odim), lambda i: (0, 0)),
        ],
        out_specs=pl.BlockSpec((blk, odim), lambda i: (i, 0)),
        out_shape=jax.ShapeDtypeStruct((n, odim), jnp.float32),
    )(acc, den, b, wp1, bp1, wp2, bp2)


@functools.lru_cache(maxsize=None)
def _make_sc_layer(n, e, hdim):
    nw = _NC * _NS
    chunk = 64
    gch = e // chunk
    cpw = gch // nw
    rem = gch - cpw * nw
    blkc = 2
    for cand in range(32, 1, -1):
        if cpw % cand == 0:
            blkc = cand
            break
    nblk_i = cpw // blkc
    kk = hdim // _L
    nden = -(-n // hdim)
    rtot = -(-(n + nden) // (16 * _NS)) * 16 * _NS
    rps = rtot // _NS
    nblk = rps // 16

    mesh = plsc.VectorSubcoreMesh(core_axis_name="c", subcore_axis_name="s")

    def body(xl_hbm, xr_hbm, src_hbm, dst_hbm, att_hbm,
             raw_out,
             srcblk, dstblk, dstc, dstv8, xlv, xrv, denv, attv,
             accs, sem_g, sem_g2):
        c = lax.axis_index("c")
        s = lax.axis_index("s")
        wid = c * _NS + s
        r0 = pl.multiple_of(s * rps, 8)
        g0w = wid * cpw + jnp.minimum(wid, rem)
        bounce = denv.at[pl.ds(0, 16)]

        zt = jnp.zeros((_L,), jnp.float32)
        for r in range(16):
            for k in range(kk):
                denv[r, pl.ds(k * _L, _L)] = zt

        def zero_body(j, carry):
            rr = pl.multiple_of(r0 + j * 16, 8)
            pltpu.sync_copy(bounce, accs.at[pl.ds(rr, 16)])
            return carry

        lax.fori_loop(0, nblk, zero_body, 0)
        pltpu.sync_copy(att_hbm, attv)
        plsc.subcore_barrier()

        att_regs = [attv[pl.ds(k * _L, _L)] for k in range(kk)]
        lane0 = lax.broadcasted_iota(jnp.int32, (_L,), 0) == 0
        iota16 = lax.broadcasted_iota(jnp.int32, (_L,), 0)
        laneid = [(k * _L + iota16).astype(jnp.float32) for k in range(kk)]

        def start_gather(b, q):
            sem = sem_g if b == 0 else sem_g2
            o = pl.multiple_of(q * chunk, 8)
            cp1 = pltpu.async_copy(
                xl_hbm.at[srcblk.at[pl.ds(o, chunk)]], xlv.at[b], sem)
            cp2 = pltpu.async_copy(
                xr_hbm.at[dstblk.at[pl.ds(o, chunk)]], xrv.at[b], sem)
            return cp1, cp2

        def compute_scatter(b, q, cps):
            cps[0].wait()
            cps[1].wait()
            xlv_b, xrv_b = xlv.at[b], xrv.at[b]
            qo = pl.multiple_of(q * chunk, 8)
            for t in range(chunk // _L):
                d = dstblk[pl.ds(qo + t * _L, _L)]
                dstc[0, pl.ds(t * _L, _L)] = d
                dstv8[0, pl.ds(t * _L, _L)] = (
                    n + lax.shift_right_logical(d, 7))

            def edge_body(i, ecarry):
                xj = [xlv_b[i, pl.ds(k * _L, _L)] for k in range(kk)]
                acc = None
                for k in range(kk):
                    z = xrv_b[i, pl.ds(k * _L, _L)] + xj[k]
                    lr = jnp.maximum(z, 0.0) + 0.2 * jnp.minimum(z, 0.0)
                    t = lr * att_regs[k]
                    acc = t if acc is None else acc + t
                w = jnp.exp(_lane_sum(acc))
                grp = dstc[0, pl.ds(pl.multiple_of((i >> 4) << 4, 16), _L)]
                dvec = _lane_perm(
                    grp, jnp.full((_L,), 0, jnp.int32) + (i & 15))
                tgtf = lax.rem(dvec, hdim).astype(jnp.float32)
                for k in range(kk):
                    eqf = jnp.maximum(1.0 - jnp.abs(laneid[k] - tgtf), 0.0)
                    denv[i, pl.ds(k * _L, _L)] = w * eqf
                for k in range(kk):
                    xlv_b[i, pl.ds(k * _L, _L)] = xj[k] * w
                return ecarry

            lax.fori_loop(0, chunk, edge_body, 0)
            pltpu.sync_copy(xlv_b, accs.at[dstc.at[0]], add=True)
            pltpu.sync_copy(denv, accs.at[dstv8.at[0]], add=True)

        def run_block(base_g, nchb):
            be = pl.multiple_of(base_g * chunk, 8)
            pltpu.sync_copy(src_hbm.at[pl.ds(be, nchb * chunk)],
                            srcblk.at[pl.ds(0, nchb * chunk)])
            pltpu.sync_copy(dst_hbm.at[pl.ds(be, nchb * chunk)],
                            dstblk.at[pl.ds(0, nchb * chunk)])
            if nchb >= 3:
                s0 = 1 if nchb % 2 == 0 else 0
                if s0:
                    cps0 = start_gather(0, 0)
                    compute_scatter(0, 0, cps0)
                cps_a = start_gather(0, s0)

                def pair_body(j, carry):
                    q0 = s0 + 2 * j
                    cps_b = start_gather(1, q0 + 1)
                    compute_scatter(0, q0, cps_a)
                    cps_a2 = start_gather(0, q0 + 2)
                    compute_scatter(1, q0 + 1, cps_b)
                    return carry

                lax.fori_loop(0, (nchb - s0 - 1) // 2, pair_body, 0)
                compute_scatter(0, nchb - 1, cps_a)
            else:
                for q in range(nchb):
                    cps = start_gather(0, q)
                    compute_scatter(0, q, cps)

        def block_body(kb, carry):
            run_block(g0w + kb * blkc, blkc)
            return carry

        lax.fori_loop(0, nblk_i, block_body, 0)

        if rem:
            @pl.when(wid < rem)
            def _extra():
                run_block(g0w + cpw, 1)

        plsc.subcore_barrier()

        def out_body(j, carry):
            rr = pl.multiple_of(r0 + j * 16, 8)
            pltpu.sync_copy(accs.at[pl.ds(rr, 16)], bounce)
            pltpu.sync_copy(bounce, raw_out.at[c, pl.ds(rr, 16)])
            return carry

        lax.fori_loop(0, nblk, out_body, 0)

    kern = pl.kernel(
        body,
        out_type=jax.ShapeDtypeStruct((_NC, rtot, hdim), jnp.float32),
        mesh=mesh,
        scratch_types=[
            pltpu.VMEM((blkc * chunk,), jnp.int32),
            pltpu.VMEM((blkc * chunk,), jnp.int32),
            pltpu.VMEM((1, chunk), jnp.int32),
            pltpu.VMEM((1, chunk), jnp.int32),
            pltpu.VMEM((2, chunk, hdim), jnp.float32),
            pltpu.VMEM((2, chunk, hdim), jnp.float32),
            pltpu.VMEM((chunk, hdim), jnp.float32),
            pltpu.VMEM((hdim,), jnp.float32),
            pltpu.VMEM_SHARED((rtot, hdim), jnp.float32),
            pltpu.SemaphoreType.DMA,
            pltpu.SemaphoreType.DMA,
        ],
    )
    return kern, nden, chunk


def _sc_layer(xl, xr, src, dst, att):
    n, hdim = xl.shape
    e = src.shape[0]
    kern, nden, chunk = _make_sc_layer(n, e, hdim)
    raw = kern(xl, xr, src, dst, att)
    acc = raw[:, :n]
    den = raw[:, n:n + nden].reshape(_NC, nden * hdim)[:, :n, None]
    return acc, den


def kernel(x, edge_index, W0_src, W0_dst, att0, b0, W1_src, W1_dst, att1, b1,
           Wp1, bp1, Wp2, bp2):
    n, _ = x.shape
    src = edge_index[0]
    dst = edge_index[1]
    blk = 1000

    xl0, xr0 = _proj2(x, W0_src, W0_dst, blk)
    acc0, den0 = _sc_layer(xl0, xr0, src, dst, att0)
    xl1, xr1 = _epi_proj2(acc0, den0, b0.reshape(1, -1), W1_src, W1_dst, blk)
    acc1, den1 = _sc_layer(xl1, xr1, src, dst, att1)
    out = _final(acc1, den1, b1.reshape(1, -1), Wp1, bp1.reshape(1, -1),
                 Wp2, bp2.reshape(1, -1), blk)
    return out


# speedup vs baseline: 15.4735x; 1.3770x over previous
import functools

import jax
import jax.numpy as jnp
from jax import lax
from jax.experimental import pallas as pl
from jax.experimental.pallas import tpu as pltpu
from jax.experimental.pallas import tpu_sc as plsc

_NC = 2
_NS = 16
_L = 16

_GDN = lax.GatherDimensionNumbers(
    offset_dims=(), collapsed_slice_dims=(0,), start_index_map=(0,))


def _lane_perm(v, idx):
    return lax.gather(v, idx[:, None], _GDN, (1,),
                      mode=lax.GatherScatterMode.PROMISE_IN_BOUNDS)


def _lane_sum(v):
    iota = lax.broadcasted_iota(jnp.int32, (_L,), 0)
    for step in (1, 2, 4, 8):
        v = v + _lane_perm(v, iota ^ step)
    return v


def _proj2_body(x_ref, wa_ref, wb_ref, oa_ref, ob_ref):
    x = x_ref[...]
    oa_ref[...] = jnp.dot(x, wa_ref[...], preferred_element_type=jnp.float32)
    ob_ref[...] = jnp.dot(x, wb_ref[...], preferred_element_type=jnp.float32)


def _proj2(x, wa, wb, blk):
    n, d = x.shape
    h = wa.shape[1]
    grid = (n + blk - 1) // blk
    return pl.pallas_call(
        _proj2_body,
        grid=(grid,),
        in_specs=[
            pl.BlockSpec((blk, d), lambda i: (i, 0)),
            pl.BlockSpec((d, h), lambda i: (0, 0)),
            pl.BlockSpec((d, h), lambda i: (0, 0)),
        ],
        out_specs=[
            pl.BlockSpec((blk, h), lambda i: (i, 0)),
            pl.BlockSpec((blk, h), lambda i: (i, 0)),
        ],
        out_shape=[
            jax.ShapeDtypeStruct((n, h), jnp.float32),
            jax.ShapeDtypeStruct((n, h), jnp.float32),
        ],
    )(x, wa, wb)


def _norm_h(acc_ref, den_ref, b_ref):
    ssum = acc_ref[0] + acc_ref[1]
    den = den_ref[0] + den_ref[1]
    den = jnp.where(den > 0.0, den, 1.0)
    return jnp.maximum(ssum / den + b_ref[...], 0.0)


def _epi_proj2_body(acc_ref, den_ref, b_ref, wa_ref, wb_ref, oa_ref, ob_ref):
    h = _norm_h(acc_ref, den_ref, b_ref)
    oa_ref[...] = jnp.dot(h, wa_ref[...], preferred_element_type=jnp.float32)
    ob_ref[...] = jnp.dot(h, wb_ref[...], preferred_element_type=jnp.float32)


def _epi_proj2(acc, den, b, wa, wb, blk):
    _, n, hdim = acc.shape
    ho = wa.shape[1]
    grid = (n + blk - 1) // blk
    return pl.pallas_call(
        _epi_proj2_body,
        grid=(grid,),
        in_specs=[
            pl.BlockSpec((2, blk, hdim), lambda i: (0, i, 0)),
            pl.BlockSpec((2, blk, 1), lambda i: (0, i, 0)),
            pl.BlockSpec((1, hdim), lambda i: (0, 0)),
            pl.BlockSpec((hdim, ho), lambda i: (0, 0)),
            pl.BlockSpec((hdim, ho), lambda i: (0, 0)),
        ],
        out_specs=[
            pl.BlockSpec((blk, ho), lambda i: (i, 0)),
            pl.BlockSpec((blk, ho), lambda i: (i, 0)),
        ],
        out_shape=[
            jax.ShapeDtypeStruct((n, ho), jnp.float32),
            jax.ShapeDtypeStruct((n, ho), jnp.float32),
        ],
    )(acc, den, b, wa, wb)


def _final_body(acc_ref, den_ref, b_ref, wp1_ref, bp1_ref, wp2_ref, bp2_ref,
                o_ref):
    h = _norm_h(acc_ref, den_ref, b_ref)
    t = jnp.dot(h, wp1_ref[...], preferred_element_type=jnp.float32)
    t = t + bp1_ref[...]
    o = jnp.dot(t, wp2_ref[...], preferred_element_type=jnp.float32)
    o_ref[...] = o + bp2_ref[...]


def _final(acc, den, b, wp1, bp1, wp2, bp2, blk):
    _, n, hdim = acc.shape
    hmid = wp1.shape[1]
    odim = wp2.shape[1]
    grid = (n + blk - 1) // blk
    return pl.pallas_call(
        _final_body,
        grid=(grid,),
        in_specs=[
            pl.BlockSpec((2, blk, hdim), lambda i: (0, i, 0)),
            pl.BlockSpec((2, blk, 1), lambda i: (0, i, 0)),
            pl.BlockSpec((1, hdim), lambda i: (0, 0)),
            pl.BlockSpec((hdim, hmid), lambda i: (0, 0)),
            pl.BlockSpec((1, hmid), lambda i: (0, 0)),
            pl.BlockSpec((hmid, odim), lambda i: (0, 0)),
            pl.BlockSpec((1, odim), lambda i: (0, 0)),
        ],
        out_specs=pl.BlockSpec((blk, odim), lambda i: (i, 0)),
        out_shape=jax.ShapeDtypeStruct((n, odim), jnp.float32),
    )(acc, den, b, wp1, bp1, wp2, bp2)


@functools.lru_cache(maxsize=None)
def _make_sc_layer(n, e, hdim):
    nw = _NC * _NS
    chunk = 64
    gch = e // chunk
    cpw = gch // nw
    rem = gch - cpw * nw
    blkc = 2
    for cand in range(32, 1, -1):
        if cpw % cand == 0:
            blkc = cand
            break
    nblk_i = cpw // blkc
    kk = hdim // _L
    nden = -(-n // hdim)
    rtot = -(-(n + nden) // (16 * _NS)) * 16 * _NS
    rps = rtot // _NS
    nblk = rps // 16

    mesh = plsc.VectorSubcoreMesh(core_axis_name="c", subcore_axis_name="s")

    def body(xl_hbm, xr_hbm, src_hbm, dst_hbm, att_hbm,
             raw_out,
             srcblk, dstblk, dstc, dstv8, xlv, xrv, denv, attv,
             accs, sem_g, sem_g2, sem_sd):
        c = lax.axis_index("c")
        s = lax.axis_index("s")
        wid = c * _NS + s
        r0 = pl.multiple_of(s * rps, 8)
        g0w = wid * cpw + jnp.minimum(wid, rem)
        bounce = denv.at[pl.ds(0, 16)]

        zt = jnp.zeros((_L,), jnp.float32)
        for r in range(16):
            for k in range(kk):
                denv[r, pl.ds(k * _L, _L)] = zt

        def zero_body(j, carry):
            rr = pl.multiple_of(r0 + j * 16, 8)
            pltpu.sync_copy(bounce, accs.at[pl.ds(rr, 16)])
            return carry

        lax.fori_loop(0, nblk, zero_body, 0)
        pltpu.sync_copy(att_hbm, attv)
        plsc.subcore_barrier()

        att_regs = [attv[pl.ds(k * _L, _L)] for k in range(kk)]
        lane0 = lax.broadcasted_iota(jnp.int32, (_L,), 0) == 0
        iota16 = lax.broadcasted_iota(jnp.int32, (_L,), 0)
        laneid = [(k * _L + iota16).astype(jnp.float32) for k in range(kk)]

        def start_gather(b, q):
            sem = sem_g if b == 0 else sem_g2
            o = pl.multiple_of(q * chunk, 8)
            cp1 = pltpu.async_copy(
                xl_hbm.at[srcblk.at[pl.ds(o, chunk)]], xlv.at[b], sem)
            cp2 = pltpu.async_copy(
                xr_hbm.at[dstblk.at[pl.ds(o, chunk)]], xrv.at[b], sem)
            return cp1, cp2

        def compute_scatter(b, q, cps):
            cps[0].wait()
            cps[1].wait()
            xlv_b, xrv_b = xlv.at[b], xrv.at[b]
            qo = pl.multiple_of(q * chunk, 8)
            for t in range(chunk // _L):
                d = dstblk[pl.ds(qo + t * _L, _L)]
                dstc[0, pl.ds(t * _L, _L)] = d
                dstv8[0, pl.ds(t * _L, _L)] = (
                    n + lax.shift_right_logical(d, 7))

            def edge_body2(i):
                xj = [xlv_b[i, pl.ds(k * _L, _L)] for k in range(kk)]
                acc = None
                for k in range(kk):
                    z = xrv_b[i, pl.ds(k * _L, _L)] + xj[k]
                    lr = jnp.maximum(z, 0.0) + 0.2 * jnp.minimum(z, 0.0)
                    t = lr * att_regs[k]
                    acc = t if acc is None else acc + t
                w = jnp.exp(_lane_sum(acc))
                grp = dstc[0, pl.ds(pl.multiple_of((i >> 4) << 4, 16), _L)]
                dvec = _lane_perm(
                    grp, jnp.full((_L,), 0, jnp.int32) + (i & 15))
                tgtf = lax.rem(dvec, hdim).astype(jnp.float32)
                for k in range(kk):
                    eqf = jnp.maximum(1.0 - jnp.abs(laneid[k] - tgtf), 0.0)
                    denv[i, pl.ds(k * _L, _L)] = w * eqf
                for k in range(kk):
                    xlv_b[i, pl.ds(k * _L, _L)] = xj[k] * w

            plsc.parallel_loop(0, chunk, unroll=2)(edge_body2)
            cpd = pltpu.async_copy(denv, accs.at[dstv8.at[0]], sem_sd,
                                   add=True)
            pltpu.sync_copy(xlv_b, accs.at[dstc.at[0]], add=True)
            cpd.wait()

        def run_block(base_g, nchb):
            be = pl.multiple_of(base_g * chunk, 8)
            pltpu.sync_copy(src_hbm.at[pl.ds(be, nchb * chunk)],
                            srcblk.at[pl.ds(0, nchb * chunk)])
            pltpu.sync_copy(dst_hbm.at[pl.ds(be, nchb * chunk)],
                            dstblk.at[pl.ds(0, nchb * chunk)])
            if nchb >= 3:
                s0 = 1 if nchb % 2 == 0 else 0
                if s0:
                    cps0 = start_gather(0, 0)
                    compute_scatter(0, 0, cps0)
                cps_a = start_gather(0, s0)

                def pair_body(j, carry):
                    q0 = s0 + 2 * j
                    cps_b = start_gather(1, q0 + 1)
                    compute_scatter(0, q0, cps_a)
                    cps_a2 = start_gather(0, q0 + 2)
                    compute_scatter(1, q0 + 1, cps_b)
                    return carry

                lax.fori_loop(0, (nchb - s0 - 1) // 2, pair_body, 0)
                compute_scatter(0, nchb - 1, cps_a)
            else:
                for q in range(nchb):
                    cps = start_gather(0, q)
                    compute_scatter(0, q, cps)

        def block_body(kb, carry):
            run_block(g0w + kb * blkc, blkc)
            return carry

        lax.fori_loop(0, nblk_i, block_body, 0)

        if rem:
            @pl.when(wid < rem)
            def _extra():
                run_block(g0w + cpw, 1)

        plsc.subcore_barrier()

        def out_body(j, carry):
            rr = pl.multiple_of(r0 + j * 16, 8)
            pltpu.sync_copy(accs.at[pl.ds(rr, 16)], bounce)
            pltpu.sync_copy(bounce, raw_out.at[c, pl.ds(rr, 16)])
            return carry

        lax.fori_loop(0, nblk, out_body, 0)

    kern = pl.kernel(
        body,
        out_type=jax.ShapeDtypeStruct((_NC, rtot, hdim), jnp.float32),
        mesh=mesh,
        scratch_types=[
            pltpu.VMEM((blkc * chunk,), jnp.int32),
            pltpu.VMEM((blkc * chunk,), jnp.int32),
            pltpu.VMEM((1, chunk), jnp.int32),
            pltpu.VMEM((1, chunk), jnp.int32),
            pltpu.VMEM((2, chunk, hdim), jnp.float32),
            pltpu.VMEM((2, chunk, hdim), jnp.float32),
            pltpu.VMEM((chunk, hdim), jnp.float32),
            pltpu.VMEM((hdim,), jnp.float32),
            pltpu.VMEM_SHARED((rtot, hdim), jnp.float32),
            pltpu.SemaphoreType.DMA,
            pltpu.SemaphoreType.DMA,
            pltpu.SemaphoreType.DMA,
        ],
    )
    return kern, nden, chunk


def _sc_layer(xl, xr, src, dst, att):
    n, hdim = xl.shape
    e = src.shape[0]
    kern, nden, chunk = _make_sc_layer(n, e, hdim)
    raw = kern(xl, xr, src, dst, att)
    acc = raw[:, :n]
    den = raw[:, n:n + nden].reshape(_NC, nden * hdim)[:, :n, None]
    return acc, den


def kernel(x, edge_index, W0_src, W0_dst, att0, b0, W1_src, W1_dst, att1, b1,
           Wp1, bp1, Wp2, bp2):
    n, _ = x.shape
    src = edge_index[0]
    dst = edge_index[1]
    blk = 1000

    xl0, xr0 = _proj2(x, W0_src, W0_dst, blk)
    acc0, den0 = _sc_layer(xl0, xr0, src, dst, att0)
    xl1, xr1 = _epi_proj2(acc0, den0, b0.reshape(1, -1), W1_src, W1_dst, blk)
    acc1, den1 = _sc_layer(xl1, xr1, src, dst, att1)
    out = _final(acc1, den1, b1.reshape(1, -1), Wp1, bp1.reshape(1, -1),
                 Wp2, bp2.reshape(1, -1), blk)
    return out
